# Initial kernel scaffold; baseline (speedup 1.0000x reference)
import jax
import jax.numpy as jnp
from jax.experimental import pallas as pl


def kernel(x, codebook):
    raise NotImplementedError("write your pallas kernel here")


# speedup vs baseline: 11.6067x; 11.6067x over previous
import functools

import jax
import jax.numpy as jnp
from jax import lax
from jax.experimental import pallas as pl
from jax.experimental.pallas import tpu as pltpu
from jax.experimental.pallas import tpu_sc as plsc

_TB = 2048


def _vq_body(x_ref, cb_ref, out_ref, idx_ref, commit_ref, commit_acc):
    n_t_blocks = pl.num_programs(1)
    step = pl.program_id(0) * n_t_blocks + pl.program_id(1)
    nsteps = pl.num_programs(0) * n_t_blocks

    xblk = x_ref[0]
    n_heads, n_code, head_dim = cb_ref.shape
    tb = xblk.shape[1]

    commit_part = jnp.float32(0.0)
    overall = jnp.zeros((tb,), jnp.int32)
    for h in range(n_heads):
        xh = xblk[h * head_dim:(h + 1) * head_dim, :]
        cb = cb_ref[h]
        dots = jax.lax.dot_general(
            cb, xh, (((1,), (0,)), ((), ())),
            preferred_element_type=jnp.float32)
        c2 = jnp.sum(cb * cb, axis=1)
        x2 = jnp.sum(xh * xh, axis=0)
        dist = (x2[None, :] + c2[:, None]) - 2.0 * dots
        min_val = jnp.min(dist, axis=0, keepdims=True)
        commit_part += jnp.sum(min_val)
        onehot = (dist == min_val).astype(jnp.float32)
        iota_k = jax.lax.broadcasted_iota(jnp.int32, (n_code, tb), 0)
        idxh = jnp.min(jnp.where(dist == min_val, iota_k, n_code), axis=0)
        overall = overall + idxh * (n_code ** h)
        quant = jax.lax.dot_general(
            cb, onehot, (((0,), (0,)), ((), ())),
            preferred_element_type=jnp.float32)
        out_ref[0, h * head_dim:(h + 1) * head_dim, :] = quant

    idx_ref[0, 0, :] = overall

    @pl.when(step == 0)
    def _init():
        commit_acc[0, 0] = commit_part

    @pl.when(step > 0)
    def _accum():
        commit_acc[0, 0] += commit_part

    @pl.when(step == nsteps - 1)
    def _finalize():
        n_tok = jnp.float32(nsteps) * tb
        commit_ref[0, 0] = commit_acc[0, 0] / (n_tok * (n_heads * head_dim))


def _sc_hist_body(idx_hbm, zeros_hbm, ones_hbm, out_hbm, idx_v, ones_v, hist_sh):
    c = lax.axis_index("c")
    s = lax.axis_index("s")
    w = s * 2 + c
    pltpu.sync_copy(idx_hbm.at[w], idx_v)
    pltpu.sync_copy(ones_hbm, ones_v)

    @pl.when(s == 0)
    def _zero():
        pltpu.sync_copy(zeros_hbm, hist_sh)

    plsc.subcore_barrier()
    for j in range(8):
        pltpu.sync_copy(ones_v, hist_sh.at[idx_v.at[j]], add=True)
    plsc.subcore_barrier()

    @pl.when(s == 0)
    def _drain():
        pltpu.sync_copy(hist_sh, out_hbm.at[c])


def _sc_hist(idx3, zeros, ones):
    mesh = plsc.VectorSubcoreMesh(core_axis_name="c", subcore_axis_name="s")
    fn = functools.partial(
        pl.kernel,
        mesh=mesh,
        out_type=jax.ShapeDtypeStruct((2, 4096, 16), jnp.float32),
        scratch_types=[
            pltpu.VMEM((8, 128), jnp.int32),
            pltpu.VMEM((128, 16), jnp.float32),
            pltpu.VMEM_SHARED((4096, 16), jnp.float32),
        ],
    )(_sc_hist_body)
    return fn(idx3, zeros, ones)


def _perp_body(c2_ref, perp_ref):
    counts = c2_ref[0, :, 0:1] + c2_ref[1, :, 0:1]
    total = jnp.sum(counts)
    prob = counts / total
    perp_ref[0, 0] = jnp.exp(-jnp.sum(prob * jnp.log(prob + 1e-7)))


def kernel(x, codebook):
    n, cd, t = x.shape
    n_heads, n_code, head_dim = codebook.shape
    tb = min(_TB, t)
    grid = (n, t // tb)

    out, idx, commit = pl.pallas_call(
        _vq_body,
        grid=grid,
        in_specs=[
            pl.BlockSpec((1, cd, tb), lambda i, j: (i, 0, j)),
            pl.BlockSpec((n_heads, n_code, head_dim), lambda i, j: (0, 0, 0)),
        ],
        out_specs=[
            pl.BlockSpec((1, cd, tb), lambda i, j: (i, 0, j)),
            pl.BlockSpec((1, 1, tb), lambda i, j: (i, 0, j)),
            pl.BlockSpec(memory_space=pltpu.SMEM, block_shape=(1, 1),
                         index_map=lambda i, j: (0, 0)),
        ],
        out_shape=[
            jax.ShapeDtypeStruct((n, cd, t), jnp.float32),
            jax.ShapeDtypeStruct((n, 1, t), jnp.int32),
            jax.ShapeDtypeStruct((1, 1), jnp.float32),
        ],
        scratch_shapes=[
            pltpu.SMEM((1, 1), jnp.float32),
        ],
        compiler_params=pltpu.CompilerParams(
            dimension_semantics=("arbitrary", "arbitrary")),
    )(x, codebook)

    idx3 = idx.reshape(32, 8, 128)
    zeros = jnp.zeros((4096, 16), jnp.float32)
    ones = jnp.ones((128, 16), jnp.float32)
    counts2 = _sc_hist(idx3, zeros, ones)

    perp = pl.pallas_call(
        _perp_body,
        out_specs=pl.BlockSpec(memory_space=pltpu.SMEM),
        out_shape=jax.ShapeDtypeStruct((1, 1), jnp.float32),
    )(counts2)

    return out, commit[0, 0], perp[0, 0]


# speedup vs baseline: 16.3984x; 1.4128x over previous
import jax
import jax.numpy as jnp
from jax.experimental import pallas as pl
from jax.experimental.pallas import tpu as pltpu

_TB = 2048


def _vq_body(x_ref, cb_ref, out_ref, commit_ref, perp_ref, counts_acc, commit_acc):
    n_t_blocks = pl.num_programs(1)
    step = pl.program_id(0) * n_t_blocks + pl.program_id(1)
    nsteps = pl.num_programs(0) * n_t_blocks

    xblk = x_ref[0]
    n_heads, n_code, head_dim = cb_ref.shape
    tb = xblk.shape[1]

    onehots = []
    commit_part = jnp.float32(0.0)
    for h in range(n_heads):
        xh = xblk[h * head_dim:(h + 1) * head_dim, :]
        cb = cb_ref[h]
        dots = jax.lax.dot_general(
            cb, xh, (((1,), (0,)), ((), ())),
            preferred_element_type=jnp.float32)
        c2 = jnp.sum(cb * cb, axis=1)
        x2 = jnp.sum(xh * xh, axis=0)
        dist = (x2[None, :] + c2[:, None]) - 2.0 * dots
        min_val = jnp.min(dist, axis=0, keepdims=True)
        commit_part += jnp.sum(min_val)
        iota_k = jax.lax.broadcasted_iota(jnp.int32, (n_code, tb), 0)
        idxh = jnp.min(jnp.where(dist == min_val, iota_k, n_code), axis=0)
        onehot = (iota_k == idxh[None, :]).astype(jnp.float32)
        quant = jax.lax.dot_general(
            cb, onehot, (((0,), (0,)), ((), ())),
            preferred_element_type=jnp.float32)
        out_ref[0, h * head_dim:(h + 1) * head_dim, :] = quant
        onehots.append(onehot)

    joint = jax.lax.dot_general(
        onehots[1], onehots[0], (((1,), (1,)), ((), ())),
        preferred_element_type=jnp.float32)

    @pl.when(step == 0)
    def _init():
        counts_acc[...] = joint
        commit_acc[0, 0] = commit_part

    @pl.when(step > 0)
    def _accum():
        counts_acc[...] += joint
        commit_acc[0, 0] += commit_part

    @pl.when(step == nsteps - 1)
    def _finalize():
        n_tok = jnp.float32(nsteps) * tb
        commit_ref[0, 0] = commit_acc[0, 0] / (n_tok * (n_heads * head_dim))
        prob = counts_acc[...] / n_tok
        perp_ref[0, 0] = jnp.exp(-jnp.sum(prob * jnp.log(prob + 1e-7)))


def kernel(x, codebook):
    n, cd, t = x.shape
    n_heads, n_code, head_dim = codebook.shape
    tb = min(_TB, t)
    grid = (n, t // tb)

    out, commit, perp = pl.pallas_call(
        _vq_body,
        grid=grid,
        in_specs=[
            pl.BlockSpec((1, cd, tb), lambda i, j: (i, 0, j)),
            pl.BlockSpec((n_heads, n_code, head_dim), lambda i, j: (0, 0, 0)),
        ],
        out_specs=[
            pl.BlockSpec((1, cd, tb), lambda i, j: (i, 0, j)),
            pl.BlockSpec(memory_space=pltpu.SMEM, block_shape=(1, 1),
                         index_map=lambda i, j: (0, 0)),
            pl.BlockSpec(memory_space=pltpu.SMEM, block_shape=(1, 1),
                         index_map=lambda i, j: (0, 0)),
        ],
        out_shape=[
            jax.ShapeDtypeStruct((n, cd, t), jnp.float32),
            jax.ShapeDtypeStruct((1, 1), jnp.float32),
            jax.ShapeDtypeStruct((1, 1), jnp.float32),
        ],
        scratch_shapes=[
            pltpu.VMEM((n_code, n_code), jnp.float32),
            pltpu.SMEM((1, 1), jnp.float32),
        ],
        compiler_params=pltpu.CompilerParams(
            dimension_semantics=("arbitrary", "arbitrary")),
    )(x, codebook)
    return out, commit[0, 0], perp[0, 0]
